# Initial kernel scaffold; baseline (speedup 1.0000x reference)
import jax
import jax.numpy as jnp
from jax.experimental import pallas as pl


def kernel(x, rows, cols, values, bias):
    raise NotImplementedError("write your pallas kernel here")


# speedup vs baseline: 5.1163x; 5.1163x over previous
import functools

import jax
import jax.numpy as jnp
from jax import lax
from jax.experimental import pallas as pl
from jax.experimental.pallas import tpu as pltpu
from jax.experimental.pallas import tpu_sc as plsc

NC = 2
NS = 16
L = 16
K = 128


def _make_spmm(n_out, batch, nnz_pad):
    chunks = nnz_pad // (NC * NS * K)
    rows_per_tile = n_out // NS
    mesh = plsc.VectorSubcoreMesh(core_axis_name="c", subcore_axis_name="s",
                                  num_cores=NC, num_subcores=NS)

    @functools.partial(
        pl.kernel,
        mesh=mesh,
        compiler_params=pltpu.CompilerParams(use_tc_tiling_on_sc=False),
        out_type=jax.ShapeDtypeStruct((NC, n_out, batch), jnp.float32),
        scratch_types=[
            pltpu.VMEM_SHARED((n_out, batch), jnp.float32),
            pltpu.VMEM((K,), jnp.int32),
            pltpu.VMEM((K,), jnp.int32),
            pltpu.VMEM((K,), jnp.float32),
            pltpu.VMEM((K, batch), jnp.float32),
            pltpu.SemaphoreType.DMA,
        ],
    )
    def spmm(xt_hbm, rows_hbm, cols_hbm, vals_hbm, out_hbm,
             acc, rows_v, cols_v, vals_v, gath_v, sem):
        c = lax.axis_index("c")
        s = lax.axis_index("s")
        wid = s * NC + c

        def zero_row(k, carry):
            for h in range(batch // L):
                gath_v[k, pl.ds(h * L, L)] = jnp.zeros((L,), jnp.float32)
            return carry
        lax.fori_loop(0, K, zero_row, 0)
        for r in range(rows_per_tile // K):
            pltpu.sync_copy(gath_v,
                            acc.at[pl.ds(s * rows_per_tile + r * K, K)])
        plsc.subcore_barrier()

        def chunk_body(t, carry):
            base = pl.multiple_of((wid * chunks + t) * K, K)
            pltpu.sync_copy(rows_hbm.at[pl.ds(base, K)], rows_v)
            pltpu.sync_copy(cols_hbm.at[pl.ds(base, K)], cols_v)
            pltpu.sync_copy(vals_hbm.at[pl.ds(base, K)], vals_v)
            pltpu.async_copy(xt_hbm.at[rows_v], gath_v, sem).wait()

            def scale_group(g, carry2):
                base_k = pl.multiple_of(g * L, L)
                v16 = vals_v[pl.ds(base_k, L)]
                for j in range(L):
                    val = v16[j]
                    for h in range(batch // L):
                        gath_v[base_k + j, pl.ds(h * L, L)] = (
                            gath_v[base_k + j, pl.ds(h * L, L)] * val)
                return carry2
            lax.fori_loop(0, K // L, scale_group, 0)

            pltpu.sync_copy(gath_v, acc.at[cols_v], add=True)
            return carry
        lax.fori_loop(0, chunks, chunk_body, 0)
        plsc.subcore_barrier()

        for r in range(rows_per_tile // K):
            off = s * rows_per_tile + r * K
            pltpu.sync_copy(acc.at[pl.ds(off, K)],
                            out_hbm.at[c].at[pl.ds(off, K)])

    return spmm


def kernel(x, rows, cols, values, bias):
    if x.ndim == 1:
        x = x[None, :]
    batch = x.shape[0]
    n_out = bias.shape[0]
    bpad = (-batch) % L
    if bpad:
        x = jnp.pad(x, ((0, bpad), (0, 0)))
    nnz = rows.shape[0]
    region = NC * NS * K
    nnz_pad = ((nnz + region - 1) // region) * region
    pad = nnz_pad - nnz
    rows_p = jnp.pad(rows.astype(jnp.int32), (0, pad))
    cols_p = jnp.pad(cols.astype(jnp.int32), (0, pad))
    vals_p = jnp.pad(values.astype(jnp.float32), (0, pad))
    xt = x.T

    spmm = _make_spmm(n_out, batch + bpad, nnz_pad)
    partials = spmm(xt, rows_p, cols_p, vals_p)
    out_t = partials[0] + partials[1] + bias[:, None].astype(jnp.float32)
    return out_t.T[:batch]


# speedup vs baseline: 13.1985x; 2.5797x over previous
import functools

import jax
import jax.numpy as jnp
from jax import lax
from jax.experimental import pallas as pl
from jax.experimental.pallas import tpu as pltpu
from jax.experimental.pallas import tpu_sc as plsc

NC = 2
NS = 16
NT = NC * NS
L = 16
K = 128


def _make_spmm(n_out, batch, chunks):
    rows_per_tile = n_out // NS
    mesh = plsc.VectorSubcoreMesh(core_axis_name="c", subcore_axis_name="s",
                                  num_cores=NC, num_subcores=NS)

    @functools.partial(
        pl.kernel,
        mesh=mesh,
        compiler_params=pltpu.CompilerParams(use_tc_tiling_on_sc=False),
        out_type=jax.ShapeDtypeStruct((NC, n_out, batch), jnp.float32),
        scratch_types=[
            pltpu.VMEM_SHARED((n_out, batch), jnp.float32),
            pltpu.VMEM((chunks, K), jnp.int32),
            pltpu.VMEM((chunks, K), jnp.int32),
            pltpu.VMEM((chunks, K), jnp.float32),
            pltpu.VMEM((K, batch), jnp.float32),
            pltpu.VMEM((K, batch), jnp.float32),
            pltpu.VMEM((K, batch), jnp.float32),
            pltpu.VMEM((K, batch), jnp.float32),
            pltpu.SemaphoreType.DMA,
            pltpu.SemaphoreType.DMA,
            pltpu.SemaphoreType.DMA,
            pltpu.SemaphoreType.DMA,
            pltpu.SemaphoreType.DMA,
        ],
    )
    def spmm(xt_hbm, rows_hbm, cols_hbm, vals_hbm, out_hbm,
             acc, rows_all, cols_all, vals_all, gath0, gath1, con0, con1,
             isem, gsem0, gsem1, ssem0, ssem1):
        c = lax.axis_index("c")
        s = lax.axis_index("s")
        wid = s * NC + c

        pltpu.async_copy(rows_hbm.at[wid], rows_all, isem)
        pltpu.async_copy(cols_hbm.at[wid], cols_all, isem)
        pltpu.async_copy(vals_hbm.at[wid], vals_all, isem)

        def zero_row(k, carry):
            for h in range(batch // L):
                gath0[k, pl.ds(h * L, L)] = jnp.zeros((L,), jnp.float32)
            return carry
        lax.fori_loop(0, K, zero_row, 0)
        for r in range(rows_per_tile // K):
            pltpu.sync_copy(gath0,
                            acc.at[pl.ds(s * rows_per_tile + r * K, K)])

        for _ in range(3):
            pltpu.make_async_copy(rows_hbm.at[wid], rows_all, isem).wait()

        pltpu.async_copy(xt_hbm.at[rows_all.at[0]], gath0, gsem0)
        pltpu.async_copy(xt_hbm.at[rows_all.at[1]], gath1, gsem1)
        plsc.subcore_barrier()

        def scale(t, gsrc, cdst):
            def scale_group(gi, carry):
                bk = pl.multiple_of(gi * L, L)
                v16 = vals_all[t, pl.ds(bk, L)]
                for j in range(L):
                    val = v16[j]
                    for h in range(batch // L):
                        cdst[bk + j, pl.ds(h * L, L)] = (
                            gsrc[bk + j, pl.ds(h * L, L)] * val)
                return carry
            lax.fori_loop(0, K // L, scale_group, 0)

        def half_step(t, gbuf, cbuf, gsem, ssem):
            pltpu.make_async_copy(xt_hbm.at[rows_all.at[t]], gbuf, gsem).wait()

            @pl.when(t >= 2)
            def _():
                pltpu.make_async_copy(
                    cbuf, acc.at[cols_all.at[t]], ssem).wait()

            scale(t, gbuf, cbuf)

            @pl.when(t + 2 < chunks)
            def _():
                pltpu.async_copy(xt_hbm.at[rows_all.at[t + 2]], gbuf, gsem)

            pltpu.async_copy(cbuf, acc.at[cols_all.at[t]], ssem, add=True)

        def pipe_body(g, carry):
            t0 = g * 2
            half_step(t0, gath0, con0, gsem0, ssem0)
            half_step(t0 + 1, gath1, con1, gsem1, ssem1)
            return carry
        lax.fori_loop(0, chunks // 2, pipe_body, 0)

        pltpu.make_async_copy(con0, acc.at[cols_all.at[0]], ssem0).wait()
        pltpu.make_async_copy(con1, acc.at[cols_all.at[1]], ssem1).wait()
        plsc.subcore_barrier()

        for r in range(rows_per_tile // K):
            off = s * rows_per_tile + r * K
            pltpu.sync_copy(acc.at[pl.ds(off, K)],
                            out_hbm.at[c].at[pl.ds(off, K)])

    return spmm


def kernel(x, rows, cols, values, bias):
    if x.ndim == 1:
        x = x[None, :]
    batch = x.shape[0]
    n_out = bias.shape[0]
    bpad = (-batch) % L
    if bpad:
        x = jnp.pad(x, ((0, bpad), (0, 0)))
    nnz = rows.shape[0]
    region = NT * K * 2
    nnz_pad = ((nnz + region - 1) // region) * region
    pad = nnz_pad - nnz
    chunks = nnz_pad // (NT * K)
    rows_p = jnp.pad(rows.astype(jnp.int32), (0, pad)).reshape(NT, chunks, K)
    cols_p = jnp.pad(cols.astype(jnp.int32), (0, pad)).reshape(NT, chunks, K)
    vals_p = jnp.pad(values.astype(jnp.float32), (0, pad)).reshape(
        NT, chunks, K)
    xt = x.T

    spmm = _make_spmm(n_out, batch + bpad, chunks)
    partials = spmm(xt, rows_p, cols_p, vals_p)
    out_t = partials[0] + partials[1] + bias[:, None].astype(jnp.float32)
    return out_t.T[:batch]


# speedup vs baseline: 17.7271x; 1.3431x over previous
import functools

import numpy as np
import jax
import jax.numpy as jnp
from jax import lax
from jax.experimental import pallas as pl
from jax.experimental.pallas import tpu as pltpu
from jax.experimental.pallas import tpu_sc as plsc

NC = 2
NS = 16
NT = NC * NS
L = 16
K = 128


def _make_spmm(n_out, batch, chunks):
    rows_per_tile = n_out // NS
    mesh = plsc.VectorSubcoreMesh(core_axis_name="c", subcore_axis_name="s",
                                  num_cores=NC, num_subcores=NS)

    @functools.partial(
        pl.kernel,
        mesh=mesh,
        compiler_params=pltpu.CompilerParams(use_tc_tiling_on_sc=False),
        out_type=jax.ShapeDtypeStruct((NC, n_out, batch), jnp.float32),
        scratch_types=[
            pltpu.VMEM_SHARED((n_out, batch), jnp.float32),
            pltpu.VMEM((chunks, K), jnp.int32),
            pltpu.VMEM((chunks, K), jnp.int32),
            pltpu.VMEM((chunks, K), jnp.float32),
            pltpu.VMEM((K, batch), jnp.float32),
            pltpu.VMEM((K, batch), jnp.float32),
            pltpu.VMEM((K, batch), jnp.float32),
            pltpu.VMEM((K, batch), jnp.float32),
            pltpu.SemaphoreType.DMA,
            pltpu.SemaphoreType.DMA,
            pltpu.SemaphoreType.DMA,
            pltpu.SemaphoreType.DMA,
            pltpu.SemaphoreType.DMA,
        ],
    )
    def spmm(xt_hbm, rows_hbm, cols_hbm, vals_hbm, out_hbm,
             acc, rows_all, cols_all, vals_all, gath0, gath1, con0, con1,
             isem, gsem0, gsem1, ssem0, ssem1):
        c = lax.axis_index("c")
        s = lax.axis_index("s")
        wid = s * NC + c

        pltpu.async_copy(rows_hbm.at[wid], rows_all, isem)
        pltpu.async_copy(cols_hbm.at[wid], cols_all, isem)
        pltpu.async_copy(vals_hbm.at[wid], vals_all, isem)

        def zero_row(k, carry):
            for h in range(batch // L):
                gath0[k, pl.ds(h * L, L)] = jnp.zeros((L,), jnp.float32)
            return carry
        lax.fori_loop(0, K, zero_row, 0)
        for r in range(rows_per_tile // K):
            pltpu.sync_copy(gath0,
                            acc.at[pl.ds(s * rows_per_tile + r * K, K)])

        for _ in range(3):
            pltpu.make_async_copy(rows_hbm.at[wid], rows_all, isem).wait()

        pltpu.async_copy(xt_hbm.at[rows_all.at[0]], gath0, gsem0)
        pltpu.async_copy(xt_hbm.at[rows_all.at[1]], gath1, gsem1)
        plsc.subcore_barrier()

        def scale(t, gsrc, cdst):
            def scale_group(gi, carry):
                bk = pl.multiple_of(gi * L, L)
                v16 = vals_all[t, pl.ds(bk, L)]
                for j in range(L):
                    val = v16[j]
                    for h in range(batch // L):
                        cdst[bk + j, pl.ds(h * L, L)] = (
                            gsrc[bk + j, pl.ds(h * L, L)] * val)
                return carry
            lax.fori_loop(0, K // L, scale_group, 0)

        def half_step(t, gbuf, cbuf, gsem, ssem):
            pltpu.make_async_copy(xt_hbm.at[rows_all.at[t]], gbuf, gsem).wait()

            @pl.when(t >= 2)
            def _():
                pltpu.make_async_copy(
                    cbuf, acc.at[cols_all.at[t]], ssem).wait()

            scale(t, gbuf, cbuf)

            @pl.when(t + 2 < chunks)
            def _():
                pltpu.async_copy(xt_hbm.at[rows_all.at[t + 2]], gbuf, gsem)

            pltpu.async_copy(cbuf, acc.at[cols_all.at[t]], ssem, add=True)

        def pipe_body(g, carry):
            t0 = g * 2
            half_step(t0, gath0, con0, gsem0, ssem0)
            half_step(t0 + 1, gath1, con1, gsem1, ssem1)
            return carry
        lax.fori_loop(0, chunks // 2, pipe_body, 0)

        pltpu.make_async_copy(con0, acc.at[cols_all.at[0]], ssem0).wait()
        pltpu.make_async_copy(con1, acc.at[cols_all.at[1]], ssem1).wait()
        plsc.subcore_barrier()

        for r in range(rows_per_tile // K):
            off = s * rows_per_tile + r * K
            pltpu.sync_copy(acc.at[pl.ds(off, K)],
                            out_hbm.at[c].at[pl.ds(off, K)])

    return spmm


def kernel(x, rows, cols, values, bias):
    if x.ndim == 1:
        x = x[None, :]
    batch = x.shape[0]
    n_out = bias.shape[0]
    bpad = (-batch) % L
    if bpad:
        x = jnp.pad(x, ((0, bpad), (0, 0)))
    nnz = rows.shape[0]
    n_in = x.shape[1]
    region = NT * K * 2
    nnz_pad = ((nnz + region - 1) // region) * region
    pad = nnz_pad - nnz
    chunks = nnz_pad // (NT * K)
    pad_rows = jnp.asarray(np.arange(pad, dtype=np.int32) * 61 % n_in)
    pad_cols = jnp.asarray(np.arange(pad, dtype=np.int32) * 61 % n_out)
    pad_vals = jnp.zeros((pad,), jnp.float32)
    rows_p = jnp.concatenate(
        [rows.astype(jnp.int32), pad_rows]).reshape(NT, chunks, K)
    cols_p = jnp.concatenate(
        [cols.astype(jnp.int32), pad_cols]).reshape(NT, chunks, K)
    vals_p = jnp.concatenate(
        [values.astype(jnp.float32), pad_vals]).reshape(NT, chunks, K)
    xt = x.T

    spmm = _make_spmm(n_out, batch + bpad, chunks)
    partials = spmm(xt, rows_p, cols_p, vals_p)
    out_t = partials[0] + partials[1] + bias[:, None].astype(jnp.float32)
    return out_t.T[:batch]
